# baseline (speedup 1.0000x reference)
import functools

import jax
import jax.numpy as jnp
from jax import lax
from jax.experimental import pallas as pl
from jax.experimental.pallas import tpu as pltpu
from jax.experimental.pallas import tpu_sc as plsc

N_DIMS = 64
SEQ_LEN = 200
BATCH = 4096
NUM_CORES = 2
NUM_SUBCORES = 16
NUM_WORKERS = NUM_CORES * NUM_SUBCORES
BTILES = BATCH // 128
LANES = 16


def kernel(categories, category_table, position_table):
    mesh = plsc.VectorSubcoreMesh(core_axis_name="c", subcore_axis_name="s")

    @functools.partial(
        pl.kernel,
        mesh=mesh,
        compiler_params=pltpu.CompilerParams(
            use_tc_tiling_on_sc=False, needs_layout_passes=False),
        out_type=jax.ShapeDtypeStruct((SEQ_LEN, 8, BTILES, 8, 128), jnp.float32),
        scratch_types=[
            pltpu.VMEM((128, SEQ_LEN), jnp.int32),
            pltpu.VMEM((SEQ_LEN, N_DIMS), jnp.float32),
            [pltpu.VMEM((128,), jnp.int32) for _ in range(2)],
            [pltpu.VMEM((128, N_DIMS), jnp.float32) for _ in range(2)],
            [pltpu.VMEM((N_DIMS, 128), jnp.float32) for _ in range(2)],
            [pltpu.SemaphoreType.DMA for _ in range(2)],
            [pltpu.SemaphoreType.DMA for _ in range(2)],
        ],
    )
    def emb_kernel(cat_hbm, table_hbm, pos_hbm, out_hbm,
                   idx_all, pos_v, idxcol, rows, obuf, gsem, wsem):
        bt = lax.axis_index("s") * NUM_CORES + lax.axis_index("c")
        pltpu.sync_copy(cat_hbm.at[pl.ds(bt * 128, 128)], idx_all)
        pltpu.sync_copy(pos_hbm, pos_v)

        lane = lax.iota(jnp.int32, LANES)
        bvec = [bg * LANES + lane for bg in range(8)]

        def extract_col(l, t):
            lsplat = jnp.full((LANES,), l, jnp.int32)
            for bg in range(8):
                col = plsc.load_gather(idx_all, [bvec[bg], lsplat])
                idxcol[t][pl.ds(bg * LANES, LANES)] = col

        def gather(l, t):
            pltpu.async_copy(table_hbm.at[idxcol[t]], rows[t], gsem[t])

        def gather_wait(l, t):
            pltpu.make_async_copy(table_hbm.at[idxcol[t]], rows[t], gsem[t]).wait()

        def write(l, t):
            for dg in range(8):
                pltpu.async_copy(obuf[t].at[pl.ds(dg * 8, 8)],
                                 out_hbm.at[l, dg, bt], wsem[t])

        def write_wait(l, t):
            for dg in range(8):
                pltpu.make_async_copy(obuf[t].at[pl.ds(dg * 8, 8)],
                                      out_hbm.at[l, dg, bt], wsem[t]).wait()

        def compute(l, t):
            lsplat = jnp.full((LANES,), l, jnp.int32)

            def d_body(d, carry):
                dsplat = jnp.full((LANES,), d, jnp.int32)
                p = plsc.load_gather(pos_v, [lsplat, dsplat])
                for bg in range(8):
                    v = plsc.load_gather(rows[t], [bvec[bg], dsplat])
                    obuf[t][d, pl.ds(bg * LANES, LANES)] = v + p
                return carry

            lax.fori_loop(0, N_DIMS, d_body, 0)

        extract_col(0, 0)
        gather(0, 0)

        def body(j, carry):
            for t in range(2):
                l = j * 2 + t
                nxt = l + 1
                @pl.when(nxt < SEQ_LEN)
                def _():
                    extract_col(nxt, 1 - t)
                    gather(nxt, 1 - t)
                gather_wait(l, t)
                @pl.when(l >= 2)
                def _():
                    write_wait(l - 2, t)
                compute(l, t)
                write(l, t)
            return carry

        lax.fori_loop(0, SEQ_LEN // 2, body, 0)
        write_wait(SEQ_LEN - 2, 0)
        write_wait(SEQ_LEN - 1, 1)

    out5d = emb_kernel(categories, category_table, position_table)
    return out5d.transpose(2, 4, 0, 1, 3).reshape(BATCH, SEQ_LEN, N_DIMS)


# speedup vs baseline: 1.4114x; 1.4114x over previous
import functools

import jax
import jax.numpy as jnp
from jax import lax
from jax.experimental import pallas as pl
from jax.experimental.pallas import tpu as pltpu
from jax.experimental.pallas import tpu_sc as plsc

N_DIMS = 64
SEQ_LEN = 200
BATCH = 4096
NUM_CORES = 2
NUM_SUBCORES = 16
NUM_WORKERS = NUM_CORES * NUM_SUBCORES
BTILES = BATCH // 128
LANES = 16


def kernel(categories, category_table, position_table):
    mesh = plsc.VectorSubcoreMesh(core_axis_name="c", subcore_axis_name="s")

    @functools.partial(
        pl.kernel,
        mesh=mesh,
        compiler_params=pltpu.CompilerParams(
            use_tc_tiling_on_sc=False, needs_layout_passes=False),
        out_type=jax.ShapeDtypeStruct((SEQ_LEN, 8, BTILES, 8, 128), jnp.float32),
        scratch_types=[
            pltpu.VMEM((128, SEQ_LEN), jnp.int32),
            pltpu.VMEM((SEQ_LEN, N_DIMS), jnp.float32),
            [pltpu.VMEM((128,), jnp.int32) for _ in range(2)],
            [pltpu.VMEM((128, N_DIMS), jnp.float32) for _ in range(2)],
            [pltpu.VMEM((N_DIMS, 128), jnp.float32) for _ in range(2)],
            [pltpu.SemaphoreType.DMA for _ in range(2)],
            [pltpu.SemaphoreType.DMA for _ in range(2)],
        ],
    )
    def emb_kernel(cat_hbm, table_hbm, pos_hbm, out_hbm,
                   idx_all, pos_v, idxcol, rows, obuf, gsem, wsem):
        bt = lax.axis_index("s") * NUM_CORES + lax.axis_index("c")
        pltpu.sync_copy(cat_hbm.at[pl.ds(bt * 128, 128)], idx_all)
        pltpu.sync_copy(pos_hbm, pos_v)

        lane = lax.iota(jnp.int32, LANES)
        bvec = [bg * LANES + lane for bg in range(8)]

        def extract_col(l, t):
            lsplat = jnp.full((LANES,), l, jnp.int32)
            for bg in range(8):
                col = plsc.load_gather(idx_all, [bvec[bg], lsplat])
                idxcol[t][pl.ds(bg * LANES, LANES)] = col

        def gather(l, t):
            pltpu.async_copy(table_hbm.at[idxcol[t]], rows[t], gsem[t])

        def gather_wait(l, t):
            pltpu.make_async_copy(table_hbm.at[idxcol[t]], rows[t], gsem[t]).wait()

        def write(l, t):
            for dg in range(8):
                pltpu.async_copy(obuf[t].at[pl.ds(dg * 8, 8)],
                                 out_hbm.at[l, dg, bt], wsem[t])

        def write_wait(l, t):
            for dg in range(8):
                pltpu.make_async_copy(obuf[t].at[pl.ds(dg * 8, 8)],
                                      out_hbm.at[l, dg, bt], wsem[t]).wait()

        def compute(l, t):
            lsplat = jnp.full((LANES,), l, jnp.int32)

            @plsc.parallel_loop(0, N_DIMS, unroll=8)
            def d_body(d):
                dsplat = jnp.full((LANES,), d, jnp.int32)
                p = plsc.load_gather(pos_v, [lsplat, dsplat])
                for bg in range(8):
                    v = plsc.load_gather(rows[t], [bvec[bg], dsplat])
                    obuf[t][d, pl.ds(bg * LANES, LANES)] = v + p

        extract_col(0, 0)
        gather(0, 0)

        def body(j, carry):
            for t in range(2):
                l = j * 2 + t
                nxt = l + 1
                @pl.when(nxt < SEQ_LEN)
                def _():
                    extract_col(nxt, 1 - t)
                    gather(nxt, 1 - t)
                gather_wait(l, t)
                @pl.when(l >= 2)
                def _():
                    write_wait(l - 2, t)
                compute(l, t)
                write(l, t)
            return carry

        lax.fori_loop(0, SEQ_LEN // 2, body, 0)
        write_wait(SEQ_LEN - 2, 0)
        write_wait(SEQ_LEN - 1, 1)

    out5d = emb_kernel(categories, category_table, position_table)
    return out5d.transpose(2, 4, 0, 1, 3).reshape(BATCH, SEQ_LEN, N_DIMS)


# speedup vs baseline: 1.4284x; 1.0121x over previous
import functools

import jax
import jax.numpy as jnp
from jax import lax
from jax.experimental import pallas as pl
from jax.experimental.pallas import tpu as pltpu
from jax.experimental.pallas import tpu_sc as plsc

N_DIMS = 64
SEQ_LEN = 200
BATCH = 4096
NUM_CORES = 2
NUM_SUBCORES = 16
NUM_WORKERS = NUM_CORES * NUM_SUBCORES
BTILES = BATCH // 128
LANES = 16


def kernel(categories, category_table, position_table):
    mesh = plsc.VectorSubcoreMesh(core_axis_name="c", subcore_axis_name="s")

    @functools.partial(
        pl.kernel,
        mesh=mesh,
        compiler_params=pltpu.CompilerParams(
            use_tc_tiling_on_sc=False, needs_layout_passes=False),
        out_type=jax.ShapeDtypeStruct((SEQ_LEN, 8, BTILES, 8, 128), jnp.float32),
        scratch_types=[
            pltpu.VMEM((128, SEQ_LEN), jnp.int32),
            pltpu.VMEM((SEQ_LEN, N_DIMS), jnp.float32),
            [pltpu.VMEM((128,), jnp.int32) for _ in range(2)],
            [pltpu.VMEM((128, N_DIMS), jnp.float32) for _ in range(2)],
            [pltpu.VMEM((N_DIMS, 128), jnp.float32) for _ in range(2)],
            [pltpu.SemaphoreType.DMA for _ in range(2)],
            [pltpu.SemaphoreType.DMA for _ in range(2)],
        ],
    )
    def emb_kernel(cat_hbm, table_hbm, pos_hbm, out_hbm,
                   idx_all, pos_v, idxcol, rows, obuf, gsem, wsem):
        bt = lax.axis_index("s") * NUM_CORES + lax.axis_index("c")
        pltpu.sync_copy(cat_hbm.at[pl.ds(bt * 128, 128)], idx_all)
        pltpu.sync_copy(pos_hbm, pos_v)

        lane = lax.iota(jnp.int32, LANES)
        bvec = [bg * LANES + lane for bg in range(8)]

        def extract_col(l, t):
            lsplat = jnp.full((LANES,), l, jnp.int32)
            for bg in range(8):
                col = plsc.load_gather(idx_all, [bvec[bg], lsplat])
                idxcol[t][pl.ds(bg * LANES, LANES)] = col

        def gather(l, t):
            pltpu.async_copy(table_hbm.at[idxcol[t]], rows[t], gsem[t])

        def gather_wait(l, t):
            pltpu.make_async_copy(table_hbm.at[idxcol[t]], rows[t], gsem[t]).wait()

        def write(l, t):
            for dg in range(8):
                pltpu.async_copy(obuf[t].at[pl.ds(dg * 8, 8)],
                                 out_hbm.at[l, dg, bt], wsem[t])

        def write_wait(l, t):
            for dg in range(8):
                pltpu.make_async_copy(obuf[t].at[pl.ds(dg * 8, 8)],
                                      out_hbm.at[l, dg, bt], wsem[t]).wait()

        dvec = [LANES * q + lane for q in range(4)]

        def compute(l, t):
            pos_q = [pos_v[l, pl.ds(LANES * q, LANES)] for q in range(4)]

            @plsc.parallel_loop(0, 128, unroll=8)
            def b_body(b):
                bsplat = jnp.full((LANES,), b, jnp.int32)
                for q in range(4):
                    v = rows[t][b, pl.ds(LANES * q, LANES)]
                    plsc.store_scatter(obuf[t], [dvec[q], bsplat], v + pos_q[q])

        extract_col(0, 0)
        gather(0, 0)

        def body(j, carry):
            for t in range(2):
                l = j * 2 + t
                nxt = l + 1
                @pl.when(nxt < SEQ_LEN)
                def _():
                    extract_col(nxt, 1 - t)
                    gather(nxt, 1 - t)
                gather_wait(l, t)
                @pl.when(l >= 2)
                def _():
                    write_wait(l - 2, t)
                compute(l, t)
                write(l, t)
            return carry

        lax.fori_loop(0, SEQ_LEN // 2, body, 0)
        write_wait(SEQ_LEN - 2, 0)
        write_wait(SEQ_LEN - 1, 1)

    out5d = emb_kernel(categories, category_table, position_table)
    return out5d.transpose(2, 4, 0, 1, 3).reshape(BATCH, SEQ_LEN, N_DIMS)


# speedup vs baseline: 1.4509x; 1.0158x over previous
import functools

import jax
import jax.numpy as jnp
from jax import lax
from jax.experimental import pallas as pl
from jax.experimental.pallas import tpu as pltpu
from jax.experimental.pallas import tpu_sc as plsc

N_DIMS = 64
SEQ_LEN = 200
BATCH = 4096
NUM_CORES = 2
NUM_SUBCORES = 16
NUM_WORKERS = NUM_CORES * NUM_SUBCORES
BTILES = BATCH // 128
LANES = 16
LPAIRS = SEQ_LEN // 2


def kernel(categories, category_table, position_table):
    mesh = plsc.VectorSubcoreMesh(core_axis_name="c", subcore_axis_name="s")

    @functools.partial(
        pl.kernel,
        mesh=mesh,
        compiler_params=pltpu.CompilerParams(
            use_tc_tiling_on_sc=False, needs_layout_passes=False),
        out_type=jax.ShapeDtypeStruct((SEQ_LEN, 8, BTILES, 8, 128), jnp.float32),
        scratch_types=[
            pltpu.VMEM((128, SEQ_LEN), jnp.int32),
            pltpu.VMEM((SEQ_LEN, N_DIMS), jnp.float32),
            [pltpu.VMEM((256,), jnp.int32) for _ in range(2)],
            [pltpu.VMEM((256, N_DIMS), jnp.float32) for _ in range(2)],
            [pltpu.VMEM((2, N_DIMS, 128), jnp.float32) for _ in range(2)],
            [pltpu.SemaphoreType.DMA for _ in range(2)],
            [pltpu.SemaphoreType.DMA for _ in range(2)],
        ],
    )
    def emb_kernel(cat_hbm, table_hbm, pos_hbm, out_hbm,
                   idx_all, pos_v, idxcol, rows, obuf, gsem, wsem):
        bt = lax.axis_index("s") * NUM_CORES + lax.axis_index("c")
        pltpu.sync_copy(cat_hbm.at[pl.ds(bt * 128, 128)], idx_all)
        pltpu.sync_copy(pos_hbm, pos_v)

        lane = lax.iota(jnp.int32, LANES)
        bvec = [bg * LANES + lane for bg in range(8)]
        dvec = [LANES * q + lane for q in range(4)]

        def extract_cols(i, t):
            for li in range(2):
                lsplat = jnp.full((LANES,), 2 * i + li, jnp.int32)
                for bg in range(8):
                    col = plsc.load_gather(idx_all, [bvec[bg], lsplat])
                    idxcol[t][pl.ds(li * 128 + bg * LANES, LANES)] = col

        def gather(i, t):
            pltpu.async_copy(table_hbm.at[idxcol[t]], rows[t], gsem[t])

        def gather_wait(i, t):
            pltpu.make_async_copy(table_hbm.at[idxcol[t]], rows[t], gsem[t]).wait()

        def write(i, t):
            for li in range(2):
                for dg in range(8):
                    pltpu.async_copy(obuf[t].at[li, pl.ds(dg * 8, 8)],
                                     out_hbm.at[2 * i + li, dg, bt], wsem[t])

        def write_wait(i, t):
            for li in range(2):
                for dg in range(8):
                    pltpu.make_async_copy(obuf[t].at[li, pl.ds(dg * 8, 8)],
                                          out_hbm.at[2 * i + li, dg, bt],
                                          wsem[t]).wait()

        def compute(i, t):
            for li in range(2):
                l = 2 * i + li
                pos_q = [pos_v[l, pl.ds(LANES * q, LANES)] for q in range(4)]
                dst = obuf[t].at[li]

                @plsc.parallel_loop(0, 128, unroll=8)
                def b_body(b):
                    bsplat = jnp.full((LANES,), b, jnp.int32)
                    for q in range(4):
                        v = rows[t][li * 128 + b, pl.ds(LANES * q, LANES)]
                        plsc.store_scatter(dst, [dvec[q], bsplat], v + pos_q[q])

        extract_cols(0, 0)
        gather(0, 0)
        extract_cols(1, 1)
        gather(1, 1)

        def body(j, carry):
            for t in range(2):
                i = j * 2 + t
                gather_wait(i, t)
                @pl.when(i >= 2)
                def _():
                    write_wait(i - 2, t)
                compute(i, t)
                write(i, t)
                @pl.when(i + 2 < LPAIRS)
                def _():
                    extract_cols(i + 2, t)
                    gather(i + 2, t)
            return carry

        lax.fori_loop(0, LPAIRS // 2, body, 0)
        write_wait(LPAIRS - 2, 0)
        write_wait(LPAIRS - 1, 1)

    out5d = emb_kernel(categories, category_table, position_table)
    return out5d.transpose(2, 4, 0, 1, 3).reshape(BATCH, SEQ_LEN, N_DIMS)


# speedup vs baseline: 1.5633x; 1.0774x over previous
import functools

import jax
import jax.numpy as jnp
from jax import lax
from jax.experimental import pallas as pl
from jax.experimental.pallas import tpu as pltpu
from jax.experimental.pallas import tpu_sc as plsc

N_DIMS = 64
SEQ_LEN = 200
BATCH = 4096

NUM_CORES = 2
NUM_SUBCORES = 16
NUM_WORKERS = NUM_CORES * NUM_SUBCORES
ROWS_PER_WORKER = BATCH // NUM_WORKERS
LANES = 16
NBUF = 2


def kernel(categories, category_table, position_table):
    mesh = plsc.VectorSubcoreMesh(core_axis_name="c", subcore_axis_name="s")

    @functools.partial(
        pl.kernel,
        mesh=mesh,
        compiler_params=pltpu.CompilerParams(use_tc_tiling_on_sc=False),
        out_type=jax.ShapeDtypeStruct((BATCH, SEQ_LEN, N_DIMS), jnp.float32),
        scratch_types=[
            pltpu.VMEM((ROWS_PER_WORKER, SEQ_LEN), jnp.int32),
            pltpu.VMEM((SEQ_LEN, N_DIMS), jnp.float32),
            [pltpu.VMEM((SEQ_LEN, N_DIMS), jnp.float32) for _ in range(NBUF)],
            [pltpu.VMEM((SEQ_LEN, N_DIMS), jnp.float32) for _ in range(NBUF)],
            [pltpu.SemaphoreType.DMA for _ in range(NBUF)],
            [pltpu.SemaphoreType.DMA for _ in range(NBUF)],
        ],
    )
    def emb_kernel(cat_hbm, table_hbm, pos_hbm, out_hbm,
                   idx_all, pos_v, rows, obuf, gsem, wsem):
        wid = lax.axis_index("s") * NUM_CORES + lax.axis_index("c")
        base = wid * ROWS_PER_WORKER
        pltpu.sync_copy(cat_hbm.at[pl.ds(base, ROWS_PER_WORKER)], idx_all)
        pltpu.sync_copy(pos_hbm, pos_v)

        def gather(i, t):
            pltpu.async_copy(table_hbm.at[idx_all.at[i]], rows[t], gsem[t])

        def gather_wait(i, t):
            pltpu.make_async_copy(
                table_hbm.at[idx_all.at[i]], rows[t], gsem[t]).wait()

        def write(i, t):
            pltpu.async_copy(obuf[t], out_hbm.at[base + i], wsem[t])

        def write_wait(i, t):
            pltpu.make_async_copy(obuf[t], out_hbm.at[base + i], wsem[t]).wait()

        for t in range(NBUF):
            gather(t, t)

        def body(j, carry):
            for t in range(NBUF):
                i = j * NBUF + t
                gather_wait(i, t)

                @pl.when(i >= NBUF)
                def _():
                    write_wait(i - NBUF, t)

                def add_row(l, c):
                    for q in range(N_DIMS // LANES):
                        sl = (l, pl.ds(q * LANES, LANES))
                        obuf[t][sl] = rows[t][sl] + pos_v[sl]
                    return c

                lax.fori_loop(0, SEQ_LEN, add_row, 0)
                write(i, t)

                @pl.when(i + NBUF < ROWS_PER_WORKER)
                def _():
                    gather(i + NBUF, t)
            return carry

        lax.fori_loop(0, ROWS_PER_WORKER // NBUF, body, 0)
        for t in range(NBUF):
            write_wait(ROWS_PER_WORKER - NBUF + t, t)

    return emb_kernel(categories, category_table, position_table)
